# baseline (speedup 1.0000x reference)
import functools

import jax
import jax.numpy as jnp
from jax import lax
from jax.experimental import pallas as pl
from jax.experimental.pallas import tpu as pltpu
from jax.experimental.pallas import tpu_sc as plsc

_LANES = 16


def _sc_partials(node_embedding, indices, node_indices, pos_indices, neg_idx2d):
    n_nodes, d = node_embedding.shape
    s = node_indices.shape[0]
    m = neg_idx2d.shape[0] * neg_idx2d.shape[1]

    info = plsc.get_sparse_core_info()
    nc, ns = info.num_cores, info.num_subcores
    nw = nc * ns
    per_w = s // nw
    chunk = 128
    n_chunks = per_w // chunk
    k8 = d // _LANES

    mesh = plsc.VectorSubcoreMesh(core_axis_name="c", subcore_axis_name="s")

    @functools.partial(
        pl.kernel,
        mesh=mesh,
        out_type=(
            jax.ShapeDtypeStruct((s, _LANES), jnp.float32),
            jax.ShapeDtypeStruct((s, _LANES), jnp.float32),
        ),
        scratch_types=[
            pltpu.VMEM((chunk,), jnp.int32),
            pltpu.VMEM((chunk,), jnp.int32),
            pltpu.VMEM((chunk,), jnp.int32),
            pltpu.VMEM((chunk,), jnp.int32),
            pltpu.VMEM((chunk, d), jnp.float32),
            pltpu.VMEM((chunk, d), jnp.float32),
            pltpu.VMEM((chunk, _LANES), jnp.float32),
            pltpu.VMEM((chunk, _LANES), jnp.float32),
            pltpu.VMEM(neg_idx2d.shape, jnp.int32),
            pltpu.SemaphoreType.DMA,
        ],
    )
    def k(emb_h, idx_h, nidx_h, pidx_h, negidx_h, pos_out, neg_out,
          nidx_v, pidx_v, gn_v, gp_v, en_v, ep_v, pb_v, nb_v, negi_v, sem):
        wid = lax.axis_index("s") * nc + lax.axis_index("c")
        base0 = wid * per_w

        pltpu.sync_copy(negidx_h, negi_v)
        v = tuple(jnp.zeros((_LANES,), jnp.float32) for _ in range(k8))
        for h in range(neg_idx2d.shape[0]):
            pltpu.async_copy(idx_h.at[negi_v.at[h]], gn_v, sem).wait()
            pltpu.async_copy(emb_h.at[gn_v], en_v, sem).wait()

            def vacc(j, vs):
                return tuple(
                    vs[kk] + en_v[j, pl.ds(kk * _LANES, _LANES)]
                    for kk in range(k8)
                )

            v = lax.fori_loop(0, chunk, vacc, v)

        def chunk_body(g, carry):
            base = base0 + g * chunk
            pltpu.sync_copy(nidx_h.at[pl.ds(base, chunk)], nidx_v)
            pltpu.sync_copy(pidx_h.at[pl.ds(base, chunk)], pidx_v)
            pltpu.async_copy(idx_h.at[nidx_v], gn_v, sem).wait()
            pltpu.async_copy(idx_h.at[pidx_v], gp_v, sem).wait()
            pltpu.async_copy(emb_h.at[gn_v], en_v, sem).wait()
            pltpu.async_copy(emb_h.at[gp_v], ep_v, sem).wait()

            def row(j, c):
                e0 = en_v[j, pl.ds(0, _LANES)]
                p0 = ep_v[j, pl.ds(0, _LANES)]
                pos = e0 * p0
                neg = e0 * v[0]
                for kk in range(1, k8):
                    ek = en_v[j, pl.ds(kk * _LANES, _LANES)]
                    pk = ep_v[j, pl.ds(kk * _LANES, _LANES)]
                    pos = pos + ek * pk
                    neg = neg + ek * v[kk]
                pb_v[j, :] = pos
                nb_v[j, :] = neg
                return c

            lax.fori_loop(0, chunk, row, 0)
            pltpu.sync_copy(pb_v, pos_out.at[pl.ds(base, chunk)])
            pltpu.sync_copy(nb_v, neg_out.at[pl.ds(base, chunk)])
            return carry

        lax.fori_loop(0, n_chunks, chunk_body, 0)

    return k(node_embedding, indices, node_indices, pos_indices, neg_idx2d)


def _tc_loss(pos_partial, neg_partial):
    s = pos_partial.shape[0]
    blk = 8192
    grid = s // blk

    def body(pref, nref, oref):
        i = pl.program_id(0)

        @pl.when(i == 0)
        def _():
            oref[...] = jnp.zeros((1, 1), jnp.float32)

        ps = jnp.sum(pref[...], axis=1)
        ns = jnp.sum(nref[...], axis=1)
        pos_l = jnp.sum(jax.nn.softplus(-ps))
        neg_l = jnp.sum(jax.nn.softplus(ns))
        oref[...] += jnp.reshape((3.0 * pos_l + neg_l) * (1.0 / s), (1, 1))

    return pl.pallas_call(
        body,
        grid=(grid,),
        in_specs=[
            pl.BlockSpec((blk, _LANES), lambda i: (i, 0)),
            pl.BlockSpec((blk, _LANES), lambda i: (i, 0)),
        ],
        out_specs=pl.BlockSpec((1, 1), lambda i: (0, 0)),
        out_shape=jax.ShapeDtypeStruct((1, 1), jnp.float32),
    )(pos_partial, neg_partial)


@jax.jit
def kernel(node_embedding, indices, node_indices, pos_indices, neg_indices):
    neg_idx2d = neg_indices.reshape(-1, 128)
    pos_p, neg_p = _sc_partials(
        node_embedding, indices, node_indices, pos_indices, neg_idx2d)
    return _tc_loss(pos_p, neg_p).reshape(1)


# speedup vs baseline: 1.3098x; 1.3098x over previous
import functools

import jax
import jax.numpy as jnp
from jax import lax
from jax.experimental import pallas as pl
from jax.experimental.pallas import tpu as pltpu
from jax.experimental.pallas import tpu_sc as plsc

_LANES = 16
_C = 128


def _sc_partials(node_embedding, indices, nidx2d, pidx2d, neg_idx2d):
    n_nodes, d = node_embedding.shape
    s = nidx2d.shape[0] * nidx2d.shape[1]

    info = plsc.get_sparse_core_info()
    nc, ns = info.num_cores, info.num_subcores
    nw = nc * ns
    per_w = s // nw
    n_chunks = per_w // _C
    k8 = d // _LANES
    lag = 3

    mesh = plsc.VectorSubcoreMesh(core_axis_name="c", subcore_axis_name="s")

    @functools.partial(
        pl.kernel,
        mesh=mesh,
        out_type=(
            jax.ShapeDtypeStruct((s * _LANES,), jnp.float32),
            jax.ShapeDtypeStruct((s * _LANES,), jnp.float32),
        ),
        scratch_types=[
            pltpu.VMEM((n_chunks, _C), jnp.int32),
            pltpu.VMEM((n_chunks, _C), jnp.int32),
            pltpu.VMEM((n_chunks, _C), jnp.int32),
            pltpu.VMEM((n_chunks, _C), jnp.int32),
            pltpu.VMEM((_C, d), jnp.float32),
            pltpu.VMEM((_C, d), jnp.float32),
            pltpu.VMEM((_C, d), jnp.float32),
            pltpu.VMEM((_C, d), jnp.float32),
            pltpu.VMEM((_C * _LANES,), jnp.float32),
            pltpu.VMEM((_C * _LANES,), jnp.float32),
            pltpu.VMEM((_C * _LANES,), jnp.float32),
            pltpu.VMEM((_C * _LANES,), jnp.float32),
            pltpu.VMEM(neg_idx2d.shape, jnp.int32),
            pltpu.SemaphoreType.DMA,
            pltpu.SemaphoreType.DMA,
            pltpu.SemaphoreType.DMA,
            pltpu.SemaphoreType.DMA,
            pltpu.SemaphoreType.DMA,
        ],
    )
    def k(emb_h, idx_h, nidx_h, pidx_h, negidx_h, pos_out, neg_out,
          sn_v, sp_v, cn_v, cp_v, en0, en1, ep0, ep1, pb0, pb1, nb0, nb1,
          negi_v, sem_p, sem_a, sem_b, sem_o0, sem_o1):
        wid = lax.axis_index("s") * nc + lax.axis_index("c")
        base0 = wid * per_w
        row0 = wid * n_chunks

        ens = (en0, en1)
        eps = (ep0, ep1)
        pbs = (pb0, pb1)
        nbs = (nb0, nb1)
        sem_g = (sem_a, sem_b)
        sem_o = (sem_o0, sem_o1)

        pltpu.sync_copy(negidx_h, negi_v)
        v = tuple(jnp.zeros((_LANES,), jnp.float32) for _ in range(k8))
        for h in range(neg_idx2d.shape[0]):
            pltpu.async_copy(idx_h.at[negi_v.at[h]], cn_v.at[0], sem_p).wait()
            pltpu.async_copy(emb_h.at[cn_v.at[0]], en0, sem_p).wait()

            def vacc(j, vs):
                return tuple(
                    vs[kk] + en0[j, pl.ds(kk * _LANES, _LANES)]
                    for kk in range(k8)
                )

            v = lax.fori_loop(0, _C, vacc, v)

        pltpu.sync_copy(nidx_h.at[pl.ds(row0, n_chunks)], sn_v)
        pltpu.sync_copy(pidx_h.at[pl.ds(row0, n_chunks)], sp_v)

        def compose(j, carry):
            pltpu.make_async_copy(idx_h.at[sn_v.at[j]], cn_v.at[j], sem_p).start()
            pltpu.make_async_copy(idx_h.at[sp_v.at[j]], cp_v.at[j], sem_p).start()

            @pl.when(j >= lag)
            def _():
                pltpu.make_async_copy(
                    idx_h.at[sn_v.at[0]], cn_v.at[0], sem_p).wait()
                pltpu.make_async_copy(
                    idx_h.at[sp_v.at[0]], cp_v.at[0], sem_p).wait()

            return carry

        lax.fori_loop(0, n_chunks, compose, 0)
        for _ in range(lag):
            pltpu.make_async_copy(idx_h.at[sn_v.at[0]], cn_v.at[0], sem_p).wait()
            pltpu.make_async_copy(idx_h.at[sp_v.at[0]], cp_v.at[0], sem_p).wait()

        def fire_rows(g, b):
            pltpu.make_async_copy(emb_h.at[cn_v.at[g]], ens[b], sem_g[b]).start()
            pltpu.make_async_copy(emb_h.at[cp_v.at[g]], eps[b], sem_g[b]).start()

        def wait_rows(b):
            pltpu.make_async_copy(emb_h.at[cn_v.at[0]], ens[b], sem_g[b]).wait()
            pltpu.make_async_copy(emb_h.at[cp_v.at[0]], eps[b], sem_g[b]).wait()

        def drain_stores(b):
            pltpu.make_async_copy(
                pbs[b], pos_out.at[pl.ds(0, _C * _LANES)], sem_o[b]).wait()
            pltpu.make_async_copy(
                nbs[b], neg_out.at[pl.ds(0, _C * _LANES)], sem_o[b]).wait()

        def compute(g, b):
            en_v, ep_v = ens[b], eps[b]
            pb_v, nb_v = pbs[b], nbs[b]

            def row(j, c):
                e0 = en_v[j, pl.ds(0, _LANES)]
                p0 = ep_v[j, pl.ds(0, _LANES)]
                pos = e0 * p0
                neg = e0 * v[0]
                for kk in range(1, k8):
                    ek = en_v[j, pl.ds(kk * _LANES, _LANES)]
                    pk = ep_v[j, pl.ds(kk * _LANES, _LANES)]
                    pos = pos + ek * pk
                    neg = neg + ek * v[kk]
                pb_v[pl.ds(j * _LANES, _LANES)] = pos
                nb_v[pl.ds(j * _LANES, _LANES)] = neg
                return c

            lax.fori_loop(0, _C, row, 0)
            base = (base0 + g * _C) * _LANES
            pltpu.make_async_copy(
                pb_v, pos_out.at[pl.ds(base, _C * _LANES)], sem_o[b]).start()
            pltpu.make_async_copy(
                nb_v, neg_out.at[pl.ds(base, _C * _LANES)], sem_o[b]).start()

        fire_rows(0, 0)

        def body2(t, carry):
            g0 = 2 * t
            fire_rows(g0 + 1, 1)
            wait_rows(0)

            @pl.when(t > 0)
            def _():
                drain_stores(0)

            compute(g0, 0)

            @pl.when(g0 + 2 < n_chunks)
            def _():
                fire_rows(g0 + 2, 0)

            wait_rows(1)

            @pl.when(t > 0)
            def _():
                drain_stores(1)

            compute(g0 + 1, 1)
            return carry

        lax.fori_loop(0, n_chunks // 2, body2, 0)
        drain_stores(0)
        drain_stores(1)

    return k(node_embedding, indices, nidx2d, pidx2d, neg_idx2d)


def _tc_loss(pos_partial, neg_partial):
    s = pos_partial.shape[0]
    blk = 8192
    grid = s // blk

    def body(pref, nref, oref):
        i = pl.program_id(0)

        @pl.when(i == 0)
        def _():
            oref[...] = jnp.zeros((1, 1), jnp.float32)

        ps = jnp.sum(pref[...], axis=1)
        ns = jnp.sum(nref[...], axis=1)
        pos_l = jnp.sum(jax.nn.softplus(-ps))
        neg_l = jnp.sum(jax.nn.softplus(ns))
        oref[...] += jnp.reshape((3.0 * pos_l + neg_l) * (1.0 / s), (1, 1))

    return pl.pallas_call(
        body,
        grid=(grid,),
        in_specs=[
            pl.BlockSpec((blk, _LANES), lambda i: (i, 0)),
            pl.BlockSpec((blk, _LANES), lambda i: (i, 0)),
        ],
        out_specs=pl.BlockSpec((1, 1), lambda i: (0, 0)),
        out_shape=jax.ShapeDtypeStruct((1, 1), jnp.float32),
    )(pos_partial, neg_partial)


@jax.jit
def kernel(node_embedding, indices, node_indices, pos_indices, neg_indices):
    nidx2d = node_indices.reshape(-1, _C)
    pidx2d = pos_indices.reshape(-1, _C)
    neg_idx2d = neg_indices.reshape(-1, _C)
    pos_p, neg_p = _sc_partials(
        node_embedding, indices, nidx2d, pidx2d, neg_idx2d)
    s = node_indices.shape[0]
    return _tc_loss(pos_p.reshape(s, _LANES), neg_p.reshape(s, _LANES)).reshape(1)


# speedup vs baseline: 1.7183x; 1.3119x over previous
import functools

import jax
import jax.numpy as jnp
from jax import lax
from jax.experimental import pallas as pl
from jax.experimental.pallas import tpu as pltpu
from jax.experimental.pallas import tpu_sc as plsc

_LANES = 16
_C = 128


def _sc_partials(node_embedding, indices, nidx2d, pidx2d, neg_idx2d):
    n_nodes, d = node_embedding.shape
    s = nidx2d.shape[0] * nidx2d.shape[1]

    info = plsc.get_sparse_core_info()
    nc, ns = info.num_cores, info.num_subcores
    nw = nc * ns
    per_w = s // nw
    n_chunks = per_w // _C
    k8 = d // _LANES
    lag = 3

    mesh = plsc.VectorSubcoreMesh(core_axis_name="c", subcore_axis_name="s")

    @functools.partial(
        pl.kernel,
        mesh=mesh,
        out_type=(
            jax.ShapeDtypeStruct((s * _LANES // 128, 128), jnp.float32),
            jax.ShapeDtypeStruct((s * _LANES // 128, 128), jnp.float32),
        ),
        scratch_types=[
            pltpu.VMEM((n_chunks, _C), jnp.int32),
            pltpu.VMEM((n_chunks, _C), jnp.int32),
            pltpu.VMEM((n_chunks, _C), jnp.int32),
            pltpu.VMEM((n_chunks, _C), jnp.int32),
            pltpu.VMEM((_C, d), jnp.float32),
            pltpu.VMEM((_C, d), jnp.float32),
            pltpu.VMEM((_C, d), jnp.float32),
            pltpu.VMEM((_C, d), jnp.float32),
            pltpu.VMEM((_C * _LANES // 128, 128), jnp.float32),
            pltpu.VMEM((_C * _LANES // 128, 128), jnp.float32),
            pltpu.VMEM((_C * _LANES // 128, 128), jnp.float32),
            pltpu.VMEM((_C * _LANES // 128, 128), jnp.float32),
            pltpu.VMEM(neg_idx2d.shape, jnp.int32),
            pltpu.SemaphoreType.DMA,
            pltpu.SemaphoreType.DMA,
            pltpu.SemaphoreType.DMA,
            pltpu.SemaphoreType.DMA,
            pltpu.SemaphoreType.DMA,
        ],
    )
    def k(emb_h, idx_h, nidx_h, pidx_h, negidx_h, pos_out, neg_out,
          sn_v, sp_v, cn_v, cp_v, en0, en1, ep0, ep1, pb0, pb1, nb0, nb1,
          negi_v, sem_p, sem_a, sem_b, sem_o0, sem_o1):
        wid = lax.axis_index("s") * nc + lax.axis_index("c")
        base0 = wid * per_w
        row0 = wid * n_chunks

        ens = (en0, en1)
        eps = (ep0, ep1)
        pbs = (pb0, pb1)
        nbs = (nb0, nb1)
        sem_g = (sem_a, sem_b)
        sem_o = (sem_o0, sem_o1)

        pltpu.sync_copy(negidx_h, negi_v)
        v = tuple(jnp.zeros((_LANES,), jnp.float32) for _ in range(k8))
        for h in range(neg_idx2d.shape[0]):
            pltpu.async_copy(idx_h.at[negi_v.at[h]], cn_v.at[0], sem_p).wait()
            pltpu.async_copy(emb_h.at[cn_v.at[0]], en0, sem_p).wait()

            def vacc(j, vs):
                return tuple(
                    vs[kk] + en0[j, pl.ds(kk * _LANES, _LANES)]
                    for kk in range(k8)
                )

            v = lax.fori_loop(0, _C, vacc, v)

        pltpu.sync_copy(nidx_h.at[pl.ds(row0, n_chunks)], sn_v)
        pltpu.sync_copy(pidx_h.at[pl.ds(row0, n_chunks)], sp_v)

        def compose(j, carry):
            pltpu.make_async_copy(idx_h.at[sn_v.at[j]], cn_v.at[j], sem_p).start()
            pltpu.make_async_copy(idx_h.at[sp_v.at[j]], cp_v.at[j], sem_p).start()

            @pl.when(j >= lag)
            def _():
                pltpu.make_async_copy(
                    idx_h.at[sn_v.at[0]], cn_v.at[0], sem_p).wait()
                pltpu.make_async_copy(
                    idx_h.at[sp_v.at[0]], cp_v.at[0], sem_p).wait()

            return carry

        lax.fori_loop(0, n_chunks, compose, 0)
        for _ in range(lag):
            pltpu.make_async_copy(idx_h.at[sn_v.at[0]], cn_v.at[0], sem_p).wait()
            pltpu.make_async_copy(idx_h.at[sp_v.at[0]], cp_v.at[0], sem_p).wait()

        def fire_rows(g, b):
            pltpu.make_async_copy(emb_h.at[cn_v.at[g]], ens[b], sem_g[b]).start()
            pltpu.make_async_copy(emb_h.at[cp_v.at[g]], eps[b], sem_g[b]).start()

        def wait_rows(b):
            pltpu.make_async_copy(emb_h.at[cn_v.at[0]], ens[b], sem_g[b]).wait()
            pltpu.make_async_copy(emb_h.at[cp_v.at[0]], eps[b], sem_g[b]).wait()

        nrow = _C * _LANES // 128

        def drain_stores(b):
            pltpu.make_async_copy(
                pbs[b], pos_out.at[pl.ds(0, nrow)], sem_o[b]).wait()
            pltpu.make_async_copy(
                nbs[b], neg_out.at[pl.ds(0, nrow)], sem_o[b]).wait()

        def compute(g, b):
            en_v, ep_v = ens[b], eps[b]
            pb_v, nb_v = pbs[b], nbs[b]

            def row(j, c):
                e0 = en_v[j, pl.ds(0, _LANES)]
                p0 = ep_v[j, pl.ds(0, _LANES)]
                pos = e0 * p0
                neg = e0 * v[0]
                for kk in range(1, k8):
                    ek = en_v[j, pl.ds(kk * _LANES, _LANES)]
                    pk = ep_v[j, pl.ds(kk * _LANES, _LANES)]
                    pos = pos + ek * pk
                    neg = neg + ek * v[kk]
                jr = lax.shift_right_logical(j, 3)
                jc = lax.bitwise_and(j, 7) * _LANES
                pb_v[jr, pl.ds(jc, _LANES)] = pos
                nb_v[jr, pl.ds(jc, _LANES)] = neg
                return c

            lax.fori_loop(0, _C, row, 0)
            base = pl.multiple_of((base0 + g * _C) * _LANES // 128, nrow)
            pltpu.make_async_copy(
                pb_v, pos_out.at[pl.ds(base, nrow)], sem_o[b]).start()
            pltpu.make_async_copy(
                nb_v, neg_out.at[pl.ds(base, nrow)], sem_o[b]).start()

        fire_rows(0, 0)

        def body2(t, carry):
            g0 = 2 * t
            fire_rows(g0 + 1, 1)
            wait_rows(0)

            @pl.when(t > 0)
            def _():
                drain_stores(0)

            compute(g0, 0)

            @pl.when(g0 + 2 < n_chunks)
            def _():
                fire_rows(g0 + 2, 0)

            wait_rows(1)

            @pl.when(t > 0)
            def _():
                drain_stores(1)

            compute(g0 + 1, 1)
            return carry

        lax.fori_loop(0, n_chunks // 2, body2, 0)
        drain_stores(0)
        drain_stores(1)

    return k(node_embedding, indices, nidx2d, pidx2d, neg_idx2d)


def _tc_loss(pos_partial, neg_partial, s):
    rows = pos_partial.shape[0]
    blk = 2048
    grid = rows // blk
    groups = 128 // _LANES

    def body(pref, nref, oref):
        i = pl.program_id(0)

        @pl.when(i == 0)
        def _():
            oref[...] = jnp.zeros((1, 1), jnp.float32)

        p = pref[...]
        n = nref[...]
        acc = jnp.zeros((), jnp.float32)
        for kk in range(groups):
            ps = jnp.sum(p[:, kk * _LANES:(kk + 1) * _LANES], axis=1)
            ns = jnp.sum(n[:, kk * _LANES:(kk + 1) * _LANES], axis=1)
            acc += 3.0 * jnp.sum(jax.nn.softplus(-ps))
            acc += jnp.sum(jax.nn.softplus(ns))
        oref[...] += jnp.reshape(acc * (1.0 / s), (1, 1))

    return pl.pallas_call(
        body,
        grid=(grid,),
        in_specs=[
            pl.BlockSpec((blk, 128), lambda i: (i, 0)),
            pl.BlockSpec((blk, 128), lambda i: (i, 0)),
        ],
        out_specs=pl.BlockSpec((1, 1), lambda i: (0, 0)),
        out_shape=jax.ShapeDtypeStruct((1, 1), jnp.float32),
    )(pos_partial, neg_partial)


@jax.jit
def kernel(node_embedding, indices, node_indices, pos_indices, neg_indices):
    nidx2d = node_indices.reshape(-1, _C)
    pidx2d = pos_indices.reshape(-1, _C)
    neg_idx2d = neg_indices.reshape(-1, _C)
    pos_p, neg_p = _sc_partials(
        node_embedding, indices, nidx2d, pidx2d, neg_idx2d)
    s = node_indices.shape[0]
    return _tc_loss(pos_p, neg_p, s).reshape(1)


# speedup vs baseline: 2.7875x; 1.6222x over previous
import functools

import jax
import jax.numpy as jnp
from jax import lax
from jax.experimental import pallas as pl
from jax.experimental.pallas import tpu as pltpu
from jax.experimental.pallas import tpu_sc as plsc

_LANES = 16
_C = 128


def _sc_partials(node_embedding, indices, nidx2d, pidx2d, neg_idx2d):
    n_nodes, d = node_embedding.shape
    s = nidx2d.shape[0] * nidx2d.shape[1]

    info = plsc.get_sparse_core_info()
    nc, ns = info.num_cores, info.num_subcores
    nw = nc * ns
    per_w = s // nw
    n_chunks = per_w // _C
    k8 = d // _LANES
    lag = 3
    nrow = _C * _LANES // 128

    mesh = plsc.VectorSubcoreMesh(core_axis_name="c", subcore_axis_name="s")

    @functools.partial(
        pl.kernel,
        mesh=mesh,
        out_type=(
            jax.ShapeDtypeStruct((s * _LANES // 128, 128), jnp.float32),
            jax.ShapeDtypeStruct((s * _LANES // 128, 128), jnp.float32),
        ),
        scratch_types=[
            pltpu.VMEM((n_chunks, _C), jnp.int32),
            pltpu.VMEM((n_chunks, _C), jnp.int32),
            pltpu.VMEM((n_chunks, _C), jnp.int32),
            pltpu.VMEM((n_chunks, _C), jnp.int32),
            pltpu.VMEM((_C, d), jnp.float32),
            pltpu.VMEM((_C, d), jnp.float32),
            pltpu.VMEM((_C, d), jnp.float32),
            pltpu.VMEM((_C, d), jnp.float32),
            pltpu.VMEM((_C * _LANES // 128, 128), jnp.float32),
            pltpu.VMEM((_C * _LANES // 128, 128), jnp.float32),
            pltpu.VMEM((_C * _LANES // 128, 128), jnp.float32),
            pltpu.VMEM((_C * _LANES // 128, 128), jnp.float32),
            pltpu.VMEM(neg_idx2d.shape, jnp.int32),
            pltpu.SemaphoreType.DMA,
            pltpu.SemaphoreType.DMA,
            pltpu.SemaphoreType.DMA,
            pltpu.SemaphoreType.DMA,
            pltpu.SemaphoreType.DMA,
        ],
    )
    def k(emb_h, idx_h, nidx_h, pidx_h, negidx_h, pos_out, neg_out,
          sn_v, sp_v, cn_v, cp_v, en0, en1, ep0, ep1, pb0, pb1, nb0, nb1,
          negi_v, sem_p, sem_a, sem_b, sem_o0, sem_o1):
        wid = lax.axis_index("s") * nc + lax.axis_index("c")
        base0 = wid * per_w
        row0 = wid * n_chunks

        ens = (en0, en1)
        eps = (ep0, ep1)
        pbs = (pb0, pb1)
        nbs = (nb0, nb1)
        sem_g = (sem_a, sem_b)
        sem_o = (sem_o0, sem_o1)

        pltpu.sync_copy(negidx_h, negi_v)
        v = tuple(jnp.zeros((_LANES,), jnp.float32) for _ in range(k8))
        for h in range(neg_idx2d.shape[0]):
            pltpu.async_copy(idx_h.at[negi_v.at[h]], cn_v.at[0], sem_p).wait()
            pltpu.async_copy(emb_h.at[cn_v.at[0]], en0, sem_p).wait()

            def vacc(j, vs):
                return tuple(
                    vs[kk] + en0[j, pl.ds(kk * _LANES, _LANES)]
                    for kk in range(k8)
                )

            v = lax.fori_loop(0, _C, vacc, v)

        pltpu.sync_copy(nidx_h.at[pl.ds(row0, n_chunks)], sn_v)
        pltpu.sync_copy(pidx_h.at[pl.ds(row0, n_chunks)], sp_v)

        def compose(j, carry):
            pltpu.make_async_copy(idx_h.at[sn_v.at[j]], cn_v.at[j], sem_p).start()
            pltpu.make_async_copy(idx_h.at[sp_v.at[j]], cp_v.at[j], sem_p).start()

            @pl.when(j >= lag)
            def _():
                pltpu.make_async_copy(
                    idx_h.at[sn_v.at[0]], cn_v.at[0], sem_p).wait()
                pltpu.make_async_copy(
                    idx_h.at[sp_v.at[0]], cp_v.at[0], sem_p).wait()

            return carry

        lax.fori_loop(0, n_chunks, compose, 0)
        for _ in range(lag):
            pltpu.make_async_copy(idx_h.at[sn_v.at[0]], cn_v.at[0], sem_p).wait()
            pltpu.make_async_copy(idx_h.at[sp_v.at[0]], cp_v.at[0], sem_p).wait()

        def fire_rows(g, b):
            pltpu.make_async_copy(emb_h.at[cn_v.at[g]], ens[b], sem_g[b]).start()
            pltpu.make_async_copy(emb_h.at[cp_v.at[g]], eps[b], sem_g[b]).start()

        def wait_rows(b):
            pltpu.make_async_copy(emb_h.at[cn_v.at[0]], ens[b], sem_g[b]).wait()
            pltpu.make_async_copy(emb_h.at[cp_v.at[0]], eps[b], sem_g[b]).wait()

        def drain_stores(b):
            pltpu.make_async_copy(
                pbs[b], pos_out.at[pl.ds(0, nrow)], sem_o[b]).wait()
            pltpu.make_async_copy(
                nbs[b], neg_out.at[pl.ds(0, nrow)], sem_o[b]).wait()

        def compute(g, b):
            en_v, ep_v = ens[b], eps[b]
            pb_v, nb_v = pbs[b], nbs[b]

            def row(j, c):
                e0 = en_v[j, pl.ds(0, _LANES)]
                p0 = ep_v[j, pl.ds(0, _LANES)]
                pos = e0 * p0
                neg = e0 * v[0]
                for kk in range(1, k8):
                    ek = en_v[j, pl.ds(kk * _LANES, _LANES)]
                    pk = ep_v[j, pl.ds(kk * _LANES, _LANES)]
                    pos = pos + ek * pk
                    neg = neg + ek * v[kk]
                jr = lax.shift_right_logical(j, 3)
                jc = lax.bitwise_and(j, 7) * _LANES
                pb_v[jr, pl.ds(jc, _LANES)] = pos
                nb_v[jr, pl.ds(jc, _LANES)] = neg
                return c

            lax.fori_loop(0, _C, row, 0)
            base = pl.multiple_of((base0 + g * _C) * _LANES // 128, nrow)
            pltpu.make_async_copy(
                pb_v, pos_out.at[pl.ds(base, nrow)], sem_o[b]).start()
            pltpu.make_async_copy(
                nb_v, neg_out.at[pl.ds(base, nrow)], sem_o[b]).start()

        fire_rows(0, 0)

        def body2(t, carry):
            g0 = 2 * t
            fire_rows(g0 + 1, 1)
            wait_rows(0)

            @pl.when(t > 0)
            def _():
                drain_stores(0)

            compute(g0, 0)

            @pl.when(g0 + 2 < n_chunks)
            def _():
                fire_rows(g0 + 2, 0)

            wait_rows(1)

            @pl.when(t > 0)
            def _():
                drain_stores(1)

            compute(g0 + 1, 1)
            return carry

        lax.fori_loop(0, n_chunks // 2, body2, 0)
        drain_stores(0)
        drain_stores(1)

    return k(node_embedding, indices, nidx2d, pidx2d, neg_idx2d)


def _tc_loss(pos_partial, neg_partial, s):
    rows = pos_partial.shape[0]
    blk = 2048
    grid = rows // blk
    groups = 128 // _LANES

    def body(pref, nref, oref):
        i = pl.program_id(0)

        @pl.when(i == 0)
        def _():
            oref[...] = jnp.zeros((1, 1), jnp.float32)

        lane = lax.broadcasted_iota(jnp.int32, (128, groups), 0)
        grp = lax.broadcasted_iota(jnp.int32, (128, groups), 1)
        gmat = jnp.where(lane // _LANES == grp, 1.0, 0.0).astype(jnp.float32)

        ps = jax.lax.dot_general(
            pref[...], gmat, (((1,), (0,)), ((), ())),
            preferred_element_type=jnp.float32,
            precision=jax.lax.Precision.HIGHEST)
        ns = jax.lax.dot_general(
            nref[...], gmat, (((1,), (0,)), ((), ())),
            preferred_element_type=jnp.float32,
            precision=jax.lax.Precision.HIGHEST)
        acc = (3.0 * jnp.sum(jax.nn.softplus(-ps))
               + jnp.sum(jax.nn.softplus(ns)))
        oref[...] += jnp.reshape(acc * (1.0 / s), (1, 1))

    return pl.pallas_call(
        body,
        grid=(grid,),
        in_specs=[
            pl.BlockSpec((blk, 128), lambda i: (i, 0)),
            pl.BlockSpec((blk, 128), lambda i: (i, 0)),
        ],
        out_specs=pl.BlockSpec((1, 1), lambda i: (0, 0)),
        out_shape=jax.ShapeDtypeStruct((1, 1), jnp.float32),
    )(pos_partial, neg_partial)


@jax.jit
def kernel(node_embedding, indices, node_indices, pos_indices, neg_indices):
    nidx2d = node_indices.reshape(-1, _C)
    pidx2d = pos_indices.reshape(-1, _C)
    neg_idx2d = neg_indices.reshape(-1, _C)
    pos_p, neg_p = _sc_partials(
        node_embedding, indices, nidx2d, pidx2d, neg_idx2d)
    s = node_indices.shape[0]
    return _tc_loss(pos_p, neg_p, s).reshape(1)


# speedup vs baseline: 2.9654x; 1.0638x over previous
import functools

import jax
import jax.numpy as jnp
from jax import lax
from jax.experimental import pallas as pl
from jax.experimental.pallas import tpu as pltpu
from jax.experimental.pallas import tpu_sc as plsc

_LANES = 16
_C = 128


def _sc_partials(node_embedding, indices, nidx2d, pidx2d, neg_idx2d):
    n_nodes, d = node_embedding.shape
    s = nidx2d.shape[0] * nidx2d.shape[1]

    info = plsc.get_sparse_core_info()
    nc, ns = info.num_cores, info.num_subcores
    nw = nc * ns
    per_w = s // nw
    n_chunks = per_w // _C
    k8 = d // _LANES
    lag = 3
    nrow = _C * _LANES // 128

    mesh = plsc.VectorSubcoreMesh(core_axis_name="c", subcore_axis_name="s")

    @functools.partial(
        pl.kernel,
        mesh=mesh,
        out_type=(
            jax.ShapeDtypeStruct((s * _LANES // 128, 128), jnp.float32),
            jax.ShapeDtypeStruct((s * _LANES // 128, 128), jnp.float32),
        ),
        scratch_types=[
            pltpu.VMEM((n_chunks, _C), jnp.int32),
            pltpu.VMEM((n_chunks, _C), jnp.int32),
            pltpu.VMEM((n_chunks, _C), jnp.int32),
            pltpu.VMEM((n_chunks, _C), jnp.int32),
            pltpu.VMEM((_C, d), jnp.float32),
            pltpu.VMEM((_C, d), jnp.float32),
            pltpu.VMEM((_C, d), jnp.float32),
            pltpu.VMEM((_C, d), jnp.float32),
            pltpu.VMEM((_C * _LANES // 128, 128), jnp.float32),
            pltpu.VMEM((_C * _LANES // 128, 128), jnp.float32),
            pltpu.VMEM((_C * _LANES // 128, 128), jnp.float32),
            pltpu.VMEM((_C * _LANES // 128, 128), jnp.float32),
            pltpu.VMEM(neg_idx2d.shape, jnp.int32),
            pltpu.SemaphoreType.DMA,
            pltpu.SemaphoreType.DMA,
            pltpu.SemaphoreType.DMA,
            pltpu.SemaphoreType.DMA,
            pltpu.SemaphoreType.DMA,
        ],
    )
    def k(emb_h, idx_h, nidx_h, pidx_h, negidx_h, pos_out, neg_out,
          sn_v, sp_v, cn_v, cp_v, en0, en1, ep0, ep1, pb0, pb1, nb0, nb1,
          negi_v, sem_p, sem_a, sem_b, sem_o0, sem_o1):
        wid = lax.axis_index("s") * nc + lax.axis_index("c")
        base0 = wid * per_w
        row0 = wid * n_chunks

        ens = (en0, en1)
        eps = (ep0, ep1)
        pbs = (pb0, pb1)
        nbs = (nb0, nb1)
        sem_g = (sem_a, sem_b)
        sem_o = (sem_o0, sem_o1)

        pltpu.sync_copy(negidx_h, negi_v)
        v = tuple(jnp.zeros((_LANES,), jnp.float32) for _ in range(k8))
        for h in range(neg_idx2d.shape[0]):
            pltpu.async_copy(idx_h.at[negi_v.at[h]], cn_v.at[0], sem_p).wait()
            pltpu.async_copy(emb_h.at[cn_v.at[0]], en0, sem_p).wait()

            def vacc(j, vs):
                return tuple(
                    vs[kk] + en0[j, pl.ds(kk * _LANES, _LANES)]
                    for kk in range(k8)
                )

            v = lax.fori_loop(0, _C, vacc, v)

        pltpu.sync_copy(nidx_h.at[pl.ds(row0, n_chunks)], sn_v)
        pltpu.sync_copy(pidx_h.at[pl.ds(row0, n_chunks)], sp_v)

        def compose(j, carry):
            pltpu.make_async_copy(idx_h.at[sn_v.at[j]], cn_v.at[j], sem_p).start()
            pltpu.make_async_copy(idx_h.at[sp_v.at[j]], cp_v.at[j], sem_p).start()

            @pl.when(j >= lag)
            def _():
                pltpu.make_async_copy(
                    idx_h.at[sn_v.at[0]], cn_v.at[0], sem_p).wait()
                pltpu.make_async_copy(
                    idx_h.at[sp_v.at[0]], cp_v.at[0], sem_p).wait()

            return carry

        lax.fori_loop(0, n_chunks, compose, 0)
        for _ in range(lag):
            pltpu.make_async_copy(idx_h.at[sn_v.at[0]], cn_v.at[0], sem_p).wait()
            pltpu.make_async_copy(idx_h.at[sp_v.at[0]], cp_v.at[0], sem_p).wait()

        def fire_rows(g, b):
            pltpu.make_async_copy(emb_h.at[cn_v.at[g]], ens[b], sem_g[b]).start()
            pltpu.make_async_copy(emb_h.at[cp_v.at[g]], eps[b], sem_g[b]).start()

        def wait_rows(b):
            pltpu.make_async_copy(emb_h.at[cn_v.at[0]], ens[b], sem_g[b]).wait()
            pltpu.make_async_copy(emb_h.at[cp_v.at[0]], eps[b], sem_g[b]).wait()

        def drain_stores(b):
            pltpu.make_async_copy(
                pbs[b], pos_out.at[pl.ds(0, nrow)], sem_o[b]).wait()
            pltpu.make_async_copy(
                nbs[b], neg_out.at[pl.ds(0, nrow)], sem_o[b]).wait()

        def compute(g, b):
            en_v, ep_v = ens[b], eps[b]
            pb_v, nb_v = pbs[b], nbs[b]

            @plsc.parallel_loop(0, _C, 1, unroll=8)
            def row(j):
                e0 = en_v[j, pl.ds(0, _LANES)]
                p0 = ep_v[j, pl.ds(0, _LANES)]
                pos = e0 * p0
                neg = e0 * v[0]
                for kk in range(1, k8):
                    ek = en_v[j, pl.ds(kk * _LANES, _LANES)]
                    pk = ep_v[j, pl.ds(kk * _LANES, _LANES)]
                    pos = pos + ek * pk
                    neg = neg + ek * v[kk]
                jr = lax.shift_right_logical(j, 3)
                jc = lax.bitwise_and(j, 7) * _LANES
                pb_v[jr, pl.ds(jc, _LANES)] = pos
                nb_v[jr, pl.ds(jc, _LANES)] = neg
            base = pl.multiple_of((base0 + g * _C) * _LANES // 128, nrow)
            pltpu.make_async_copy(
                pb_v, pos_out.at[pl.ds(base, nrow)], sem_o[b]).start()
            pltpu.make_async_copy(
                nb_v, neg_out.at[pl.ds(base, nrow)], sem_o[b]).start()

        fire_rows(0, 0)

        def body2(t, carry):
            g0 = 2 * t
            fire_rows(g0 + 1, 1)
            wait_rows(0)

            @pl.when(t > 0)
            def _():
                drain_stores(0)

            compute(g0, 0)

            @pl.when(g0 + 2 < n_chunks)
            def _():
                fire_rows(g0 + 2, 0)

            wait_rows(1)

            @pl.when(t > 0)
            def _():
                drain_stores(1)

            compute(g0 + 1, 1)
            return carry

        lax.fori_loop(0, n_chunks // 2, body2, 0)
        drain_stores(0)
        drain_stores(1)

    return k(node_embedding, indices, nidx2d, pidx2d, neg_idx2d)


def _tc_loss(pos_partial, neg_partial, s):
    rows = pos_partial.shape[0]
    blk = 2048
    grid = rows // blk
    groups = 128 // _LANES

    def body(pref, nref, oref):
        i = pl.program_id(0)

        @pl.when(i == 0)
        def _():
            oref[...] = jnp.zeros((1, 1), jnp.float32)

        lane = lax.broadcasted_iota(jnp.int32, (128, groups), 0)
        grp = lax.broadcasted_iota(jnp.int32, (128, groups), 1)
        gmat = jnp.where(lane // _LANES == grp, 1.0, 0.0).astype(jnp.float32)

        ps = jax.lax.dot_general(
            pref[...], gmat, (((1,), (0,)), ((), ())),
            preferred_element_type=jnp.float32,
            precision=jax.lax.Precision.HIGHEST)
        ns = jax.lax.dot_general(
            nref[...], gmat, (((1,), (0,)), ((), ())),
            preferred_element_type=jnp.float32,
            precision=jax.lax.Precision.HIGHEST)
        acc = (3.0 * jnp.sum(jax.nn.softplus(-ps))
               + jnp.sum(jax.nn.softplus(ns)))
        oref[...] += jnp.reshape(acc * (1.0 / s), (1, 1))

    return pl.pallas_call(
        body,
        grid=(grid,),
        in_specs=[
            pl.BlockSpec((blk, 128), lambda i: (i, 0)),
            pl.BlockSpec((blk, 128), lambda i: (i, 0)),
        ],
        out_specs=pl.BlockSpec((1, 1), lambda i: (0, 0)),
        out_shape=jax.ShapeDtypeStruct((1, 1), jnp.float32),
    )(pos_partial, neg_partial)


@jax.jit
def kernel(node_embedding, indices, node_indices, pos_indices, neg_indices):
    nidx2d = node_indices.reshape(-1, _C)
    pidx2d = pos_indices.reshape(-1, _C)
    neg_idx2d = neg_indices.reshape(-1, _C)
    pos_p, neg_p = _sc_partials(
        node_embedding, indices, nidx2d, pidx2d, neg_idx2d)
    s = node_indices.shape[0]
    return _tc_loss(pos_p, neg_p, s).reshape(1)


# speedup vs baseline: 3.0194x; 1.0182x over previous
import functools

import jax
import jax.numpy as jnp
from jax import lax
from jax.experimental import pallas as pl
from jax.experimental.pallas import tpu as pltpu
from jax.experimental.pallas import tpu_sc as plsc

_LANES = 16
_C = 128


def _sc_partials(node_embedding, indices, nidx2d, pidx2d, neg_idx2d):
    n_nodes, d = node_embedding.shape
    s = nidx2d.shape[0] * nidx2d.shape[1]

    info = plsc.get_sparse_core_info()
    nc, ns = info.num_cores, info.num_subcores
    nw = nc * ns
    per_w = s // nw
    n_chunks = per_w // _C
    k8 = d // _LANES
    lag = 3
    nrow = _C * _LANES // 128

    mesh = plsc.VectorSubcoreMesh(core_axis_name="c", subcore_axis_name="s")

    @functools.partial(
        pl.kernel,
        mesh=mesh,
        out_type=(
            jax.ShapeDtypeStruct((s * _LANES // 128, 128), jnp.float32),
            jax.ShapeDtypeStruct((s * _LANES // 128, 128), jnp.float32),
        ),
        scratch_types=[
            pltpu.VMEM((n_chunks, _C), jnp.int32),
            pltpu.VMEM((n_chunks, _C), jnp.int32),
            pltpu.VMEM((n_chunks, _C), jnp.int32),
            pltpu.VMEM((n_chunks, _C), jnp.int32),
            pltpu.VMEM((_C, d), jnp.float32),
            pltpu.VMEM((_C, d), jnp.float32),
            pltpu.VMEM((_C, d), jnp.float32),
            pltpu.VMEM((_C, d), jnp.float32),
            pltpu.VMEM((_C * _LANES // 128, 128), jnp.float32),
            pltpu.VMEM((_C * _LANES // 128, 128), jnp.float32),
            pltpu.VMEM((_C * _LANES // 128, 128), jnp.float32),
            pltpu.VMEM((_C * _LANES // 128, 128), jnp.float32),
            pltpu.VMEM(neg_idx2d.shape, jnp.int32),
            pltpu.SemaphoreType.DMA,
            pltpu.SemaphoreType.DMA,
            pltpu.SemaphoreType.DMA,
            pltpu.SemaphoreType.DMA,
            pltpu.SemaphoreType.DMA,
        ],
    )
    def k(emb_h, idx_h, nidx_h, pidx_h, negidx_h, pos_out, neg_out,
          sn_v, sp_v, cn_v, cp_v, en0, en1, ep0, ep1, pb0, pb1, nb0, nb1,
          negi_v, sem_p, sem_a, sem_b, sem_o0, sem_o1):
        wid = lax.axis_index("s") * nc + lax.axis_index("c")
        base0 = wid * per_w
        row0 = wid * n_chunks

        ens = (en0, en1)
        eps = (ep0, ep1)
        pbs = (pb0, pb1)
        nbs = (nb0, nb1)
        sem_g = (sem_a, sem_b)
        sem_o = (sem_o0, sem_o1)

        pltpu.sync_copy(negidx_h, negi_v)
        v = tuple(jnp.zeros((_LANES,), jnp.float32) for _ in range(k8))
        for h in range(neg_idx2d.shape[0]):
            pltpu.async_copy(idx_h.at[negi_v.at[h]], cn_v.at[0], sem_p).wait()
            pltpu.async_copy(emb_h.at[cn_v.at[0]], en0, sem_p).wait()

            def vacc(j, vs):
                return tuple(
                    vs[kk] + en0[j, pl.ds(kk * _LANES, _LANES)]
                    for kk in range(k8)
                )

            v = lax.fori_loop(0, _C, vacc, v)

        pltpu.sync_copy(nidx_h.at[pl.ds(row0, n_chunks)], sn_v)
        pltpu.sync_copy(pidx_h.at[pl.ds(row0, n_chunks)], sp_v)

        def compose(j, carry):
            pltpu.make_async_copy(idx_h.at[sn_v.at[j]], cn_v.at[j], sem_p).start()
            pltpu.make_async_copy(idx_h.at[sp_v.at[j]], cp_v.at[j], sem_p).start()

            @pl.when(j >= lag)
            def _():
                pltpu.make_async_copy(
                    idx_h.at[sn_v.at[0]], cn_v.at[0], sem_p).wait()
                pltpu.make_async_copy(
                    idx_h.at[sp_v.at[0]], cp_v.at[0], sem_p).wait()

            return carry

        lax.fori_loop(0, n_chunks, compose, 0)
        for _ in range(lag):
            pltpu.make_async_copy(idx_h.at[sn_v.at[0]], cn_v.at[0], sem_p).wait()
            pltpu.make_async_copy(idx_h.at[sp_v.at[0]], cp_v.at[0], sem_p).wait()

        def fire_rows(g, b):
            pltpu.make_async_copy(emb_h.at[cn_v.at[g]], ens[b], sem_g[b]).start()
            pltpu.make_async_copy(emb_h.at[cp_v.at[g]], eps[b], sem_g[b]).start()

        def wait_rows(b):
            pltpu.make_async_copy(emb_h.at[cn_v.at[0]], ens[b], sem_g[b]).wait()
            pltpu.make_async_copy(emb_h.at[cp_v.at[0]], eps[b], sem_g[b]).wait()

        def drain_stores(b):
            pltpu.make_async_copy(
                pbs[b], pos_out.at[pl.ds(0, nrow)], sem_o[b]).wait()
            pltpu.make_async_copy(
                nbs[b], neg_out.at[pl.ds(0, nrow)], sem_o[b]).wait()

        def compute(g, b):
            en_v, ep_v = ens[b], eps[b]
            pb_v, nb_v = pbs[b], nbs[b]

            @plsc.parallel_loop(0, _C, 1, unroll=16)
            def row(j):
                e0 = en_v[j, pl.ds(0, _LANES)]
                p0 = ep_v[j, pl.ds(0, _LANES)]
                pos = e0 * p0
                neg = e0 * v[0]
                for kk in range(1, k8):
                    ek = en_v[j, pl.ds(kk * _LANES, _LANES)]
                    pk = ep_v[j, pl.ds(kk * _LANES, _LANES)]
                    pos = pos + ek * pk
                    neg = neg + ek * v[kk]
                jr = lax.shift_right_logical(j, 3)
                jc = lax.bitwise_and(j, 7) * _LANES
                pb_v[jr, pl.ds(jc, _LANES)] = pos
                nb_v[jr, pl.ds(jc, _LANES)] = neg
            base = pl.multiple_of((base0 + g * _C) * _LANES // 128, nrow)
            pltpu.make_async_copy(
                pb_v, pos_out.at[pl.ds(base, nrow)], sem_o[b]).start()
            pltpu.make_async_copy(
                nb_v, neg_out.at[pl.ds(base, nrow)], sem_o[b]).start()

        fire_rows(0, 0)

        def body2(t, carry):
            g0 = 2 * t
            fire_rows(g0 + 1, 1)
            wait_rows(0)

            @pl.when(t > 0)
            def _():
                drain_stores(0)

            compute(g0, 0)

            @pl.when(g0 + 2 < n_chunks)
            def _():
                fire_rows(g0 + 2, 0)

            wait_rows(1)

            @pl.when(t > 0)
            def _():
                drain_stores(1)

            compute(g0 + 1, 1)
            return carry

        lax.fori_loop(0, n_chunks // 2, body2, 0)
        drain_stores(0)
        drain_stores(1)

    return k(node_embedding, indices, nidx2d, pidx2d, neg_idx2d)


def _tc_loss(pos_partial, neg_partial, s):
    rows = pos_partial.shape[0]
    blk = 2048
    grid = rows // blk
    groups = 128 // _LANES

    def body(pref, nref, oref):
        i = pl.program_id(0)

        @pl.when(i == 0)
        def _():
            oref[...] = jnp.zeros((1, 1), jnp.float32)

        lane = lax.broadcasted_iota(jnp.int32, (128, groups), 0)
        grp = lax.broadcasted_iota(jnp.int32, (128, groups), 1)
        gmat = jnp.where(lane // _LANES == grp, 1.0, 0.0).astype(jnp.float32)

        ps = jax.lax.dot_general(
            pref[...], gmat, (((1,), (0,)), ((), ())),
            preferred_element_type=jnp.float32)
        ns = jax.lax.dot_general(
            nref[...], gmat, (((1,), (0,)), ((), ())),
            preferred_element_type=jnp.float32)
        acc = (3.0 * jnp.sum(jax.nn.softplus(-ps))
               + jnp.sum(jax.nn.softplus(ns)))
        oref[...] += jnp.reshape(acc * (1.0 / s), (1, 1))

    return pl.pallas_call(
        body,
        grid=(grid,),
        in_specs=[
            pl.BlockSpec((blk, 128), lambda i: (i, 0)),
            pl.BlockSpec((blk, 128), lambda i: (i, 0)),
        ],
        out_specs=pl.BlockSpec((1, 1), lambda i: (0, 0)),
        out_shape=jax.ShapeDtypeStruct((1, 1), jnp.float32),
    )(pos_partial, neg_partial)


@jax.jit
def kernel(node_embedding, indices, node_indices, pos_indices, neg_indices):
    nidx2d = node_indices.reshape(-1, _C)
    pidx2d = pos_indices.reshape(-1, _C)
    neg_idx2d = neg_indices.reshape(-1, _C)
    pos_p, neg_p = _sc_partials(
        node_embedding, indices, nidx2d, pidx2d, neg_idx2d)
    s = node_indices.shape[0]
    return _tc_loss(pos_p, neg_p, s).reshape(1)


# speedup vs baseline: 3.4118x; 1.1299x over previous
import functools

import jax
import jax.numpy as jnp
from jax import lax
from jax.experimental import pallas as pl
from jax.experimental.pallas import tpu as pltpu
from jax.experimental.pallas import tpu_sc as plsc

_LANES = 16
_C = 128


def _sc_partials(node_embedding, indices, nidx2d, pidx2d, neg_idx2d):
    n_nodes, d = node_embedding.shape
    s = nidx2d.shape[0] * nidx2d.shape[1]

    info = plsc.get_sparse_core_info()
    nc, ns = info.num_cores, info.num_subcores
    nw = nc * ns
    per_w = s // nw
    n_chunks = per_w // _C
    k8 = d // _LANES
    lag = 3
    nrow = _C * _LANES // 128

    mesh = plsc.VectorSubcoreMesh(core_axis_name="c", subcore_axis_name="s")

    @functools.partial(
        pl.kernel,
        mesh=mesh,
        out_type=(
            jax.ShapeDtypeStruct((s * _LANES // 128, 128), jnp.float32),
            jax.ShapeDtypeStruct((s * _LANES // 128, 128), jnp.float32),
        ),
        scratch_types=[
            pltpu.VMEM((n_chunks, _C), jnp.int32),
            pltpu.VMEM((n_chunks, _C), jnp.int32),
            pltpu.VMEM((n_chunks, _C), jnp.int32),
            pltpu.VMEM((n_chunks, _C), jnp.int32),
            pltpu.VMEM((_C, d), jnp.float32),
            pltpu.VMEM((_C, d), jnp.float32),
            pltpu.VMEM((_C, d), jnp.float32),
            pltpu.VMEM((_C, d), jnp.float32),
            pltpu.VMEM((_C * _LANES // 128, 128), jnp.float32),
            pltpu.VMEM((_C * _LANES // 128, 128), jnp.float32),
            pltpu.VMEM((_C * _LANES // 128, 128), jnp.float32),
            pltpu.VMEM((_C * _LANES // 128, 128), jnp.float32),
            pltpu.VMEM(neg_idx2d.shape, jnp.int32),
            pltpu.VMEM_SHARED((n_nodes,), jnp.int32),
            pltpu.SemaphoreType.DMA,
            pltpu.SemaphoreType.DMA,
            pltpu.SemaphoreType.DMA,
            pltpu.SemaphoreType.DMA,
            pltpu.SemaphoreType.DMA,
        ],
    )
    def k(emb_h, idx_h, nidx_h, pidx_h, negidx_h, pos_out, neg_out,
          sn_v, sp_v, cn_v, cp_v, en0, en1, ep0, ep1, pb0, pb1, nb0, nb1,
          negi_v, idx_sp, sem_p, sem_a, sem_b, sem_o0, sem_o1):
        wid = lax.axis_index("s") * nc + lax.axis_index("c")
        base0 = wid * per_w
        row0 = wid * n_chunks

        ens = (en0, en1)
        eps = (ep0, ep1)
        pbs = (pb0, pb1)
        nbs = (nb0, nb1)
        sem_g = (sem_a, sem_b)
        sem_o = (sem_o0, sem_o1)

        @pl.when(lax.axis_index("s") == 0)
        def _():
            pltpu.sync_copy(idx_h, idx_sp)

        plsc.subcore_barrier()

        pltpu.sync_copy(negidx_h, negi_v)
        v = tuple(jnp.zeros((_LANES,), jnp.float32) for _ in range(k8))
        for h in range(neg_idx2d.shape[0]):
            pltpu.async_copy(idx_sp.at[negi_v.at[h]], cn_v.at[0], sem_p).wait()
            pltpu.async_copy(emb_h.at[cn_v.at[0]], en0, sem_p).wait()

            def vacc(j, vs):
                return tuple(
                    vs[kk] + en0[j, pl.ds(kk * _LANES, _LANES)]
                    for kk in range(k8)
                )

            v = lax.fori_loop(0, _C, vacc, v)

        pltpu.sync_copy(nidx_h.at[pl.ds(row0, n_chunks)], sn_v)
        pltpu.sync_copy(pidx_h.at[pl.ds(row0, n_chunks)], sp_v)

        def compose(j, carry):
            pltpu.make_async_copy(idx_sp.at[sn_v.at[j]], cn_v.at[j], sem_p).start()
            pltpu.make_async_copy(idx_sp.at[sp_v.at[j]], cp_v.at[j], sem_p).start()

            @pl.when(j >= lag)
            def _():
                pltpu.make_async_copy(
                    idx_sp.at[sn_v.at[0]], cn_v.at[0], sem_p).wait()
                pltpu.make_async_copy(
                    idx_sp.at[sp_v.at[0]], cp_v.at[0], sem_p).wait()

            return carry

        lax.fori_loop(0, n_chunks, compose, 0)
        for _ in range(lag):
            pltpu.make_async_copy(idx_sp.at[sn_v.at[0]], cn_v.at[0], sem_p).wait()
            pltpu.make_async_copy(idx_sp.at[sp_v.at[0]], cp_v.at[0], sem_p).wait()

        def fire_rows(g, b):
            pltpu.make_async_copy(emb_h.at[cn_v.at[g]], ens[b], sem_g[b]).start()
            pltpu.make_async_copy(emb_h.at[cp_v.at[g]], eps[b], sem_g[b]).start()

        def wait_rows(b):
            pltpu.make_async_copy(emb_h.at[cn_v.at[0]], ens[b], sem_g[b]).wait()
            pltpu.make_async_copy(emb_h.at[cp_v.at[0]], eps[b], sem_g[b]).wait()

        def drain_stores(b):
            pltpu.make_async_copy(
                pbs[b], pos_out.at[pl.ds(0, nrow)], sem_o[b]).wait()
            pltpu.make_async_copy(
                nbs[b], neg_out.at[pl.ds(0, nrow)], sem_o[b]).wait()

        def compute(g, b):
            en_v, ep_v = ens[b], eps[b]
            pb_v, nb_v = pbs[b], nbs[b]

            @plsc.parallel_loop(0, _C, 1, unroll=16)
            def row(j):
                e0 = en_v[j, pl.ds(0, _LANES)]
                p0 = ep_v[j, pl.ds(0, _LANES)]
                pos = e0 * p0
                neg = e0 * v[0]
                for kk in range(1, k8):
                    ek = en_v[j, pl.ds(kk * _LANES, _LANES)]
                    pk = ep_v[j, pl.ds(kk * _LANES, _LANES)]
                    pos = pos + ek * pk
                    neg = neg + ek * v[kk]
                jr = lax.shift_right_logical(j, 3)
                jc = lax.bitwise_and(j, 7) * _LANES
                pb_v[jr, pl.ds(jc, _LANES)] = pos
                nb_v[jr, pl.ds(jc, _LANES)] = neg
            base = pl.multiple_of((base0 + g * _C) * _LANES // 128, nrow)
            pltpu.make_async_copy(
                pb_v, pos_out.at[pl.ds(base, nrow)], sem_o[b]).start()
            pltpu.make_async_copy(
                nb_v, neg_out.at[pl.ds(base, nrow)], sem_o[b]).start()

        fire_rows(0, 0)

        def body2(t, carry):
            g0 = 2 * t
            fire_rows(g0 + 1, 1)
            wait_rows(0)

            @pl.when(t > 0)
            def _():
                drain_stores(0)

            compute(g0, 0)

            @pl.when(g0 + 2 < n_chunks)
            def _():
                fire_rows(g0 + 2, 0)

            wait_rows(1)

            @pl.when(t > 0)
            def _():
                drain_stores(1)

            compute(g0 + 1, 1)
            return carry

        lax.fori_loop(0, n_chunks // 2, body2, 0)
        drain_stores(0)
        drain_stores(1)

    return k(node_embedding, indices, nidx2d, pidx2d, neg_idx2d)


def _tc_loss(pos_partial, neg_partial, s):
    rows = pos_partial.shape[0]
    blk = 2048
    grid = rows // blk
    groups = 128 // _LANES

    def body(pref, nref, oref):
        i = pl.program_id(0)

        @pl.when(i == 0)
        def _():
            oref[...] = jnp.zeros((1, 1), jnp.float32)

        lane = lax.broadcasted_iota(jnp.int32, (128, groups), 0)
        grp = lax.broadcasted_iota(jnp.int32, (128, groups), 1)
        gmat = jnp.where(lane // _LANES == grp, 1.0, 0.0).astype(jnp.float32)

        ps = jax.lax.dot_general(
            pref[...], gmat, (((1,), (0,)), ((), ())),
            preferred_element_type=jnp.float32)
        ns = jax.lax.dot_general(
            nref[...], gmat, (((1,), (0,)), ((), ())),
            preferred_element_type=jnp.float32)
        acc = (3.0 * jnp.sum(jax.nn.softplus(-ps))
               + jnp.sum(jax.nn.softplus(ns)))
        oref[...] += jnp.reshape(acc * (1.0 / s), (1, 1))

    return pl.pallas_call(
        body,
        grid=(grid,),
        in_specs=[
            pl.BlockSpec((blk, 128), lambda i: (i, 0)),
            pl.BlockSpec((blk, 128), lambda i: (i, 0)),
        ],
        out_specs=pl.BlockSpec((1, 1), lambda i: (0, 0)),
        out_shape=jax.ShapeDtypeStruct((1, 1), jnp.float32),
    )(pos_partial, neg_partial)


@jax.jit
def kernel(node_embedding, indices, node_indices, pos_indices, neg_indices):
    nidx2d = node_indices.reshape(-1, _C)
    pidx2d = pos_indices.reshape(-1, _C)
    neg_idx2d = neg_indices.reshape(-1, _C)
    pos_p, neg_p = _sc_partials(
        node_embedding, indices, nidx2d, pidx2d, neg_idx2d)
    s = node_indices.shape[0]
    return _tc_loss(pos_p, neg_p, s).reshape(1)


# speedup vs baseline: 3.4431x; 1.0092x over previous
import functools

import jax
import jax.numpy as jnp
from jax import lax
from jax.experimental import pallas as pl
from jax.experimental.pallas import tpu as pltpu
from jax.experimental.pallas import tpu_sc as plsc

_LANES = 16
_C = 128


def _sc_partials(node_embedding, indices, nidx2d, pidx2d, neg_idx2d):
    n_nodes, d = node_embedding.shape
    s = nidx2d.shape[0] * nidx2d.shape[1]

    info = plsc.get_sparse_core_info()
    nc, ns = info.num_cores, info.num_subcores
    nw = nc * ns
    per_w = s // nw
    n_chunks = per_w // _C
    k8 = d // _LANES
    lag = 3
    nrow = _C * _LANES // 128

    mesh = plsc.VectorSubcoreMesh(core_axis_name="c", subcore_axis_name="s")

    @functools.partial(
        pl.kernel,
        mesh=mesh,
        out_type=(
            jax.ShapeDtypeStruct((s * _LANES // 128, 128), jnp.float32),
            jax.ShapeDtypeStruct((s * _LANES // 128, 128), jnp.float32),
        ),
        scratch_types=[
            pltpu.VMEM((n_chunks, _C), jnp.int32),
            pltpu.VMEM((n_chunks, _C), jnp.int32),
            pltpu.VMEM((n_chunks, _C), jnp.int32),
            pltpu.VMEM((n_chunks, _C), jnp.int32),
            pltpu.VMEM((_C, d), jnp.float32),
            pltpu.VMEM((_C, d), jnp.float32),
            pltpu.VMEM((_C, d), jnp.float32),
            pltpu.VMEM((_C, d), jnp.float32),
            pltpu.VMEM((_C * _LANES // 128, 128), jnp.float32),
            pltpu.VMEM((_C * _LANES // 128, 128), jnp.float32),
            pltpu.VMEM((_C * _LANES // 128, 128), jnp.float32),
            pltpu.VMEM((_C * _LANES // 128, 128), jnp.float32),
            pltpu.VMEM(neg_idx2d.shape, jnp.int32),
            pltpu.VMEM_SHARED((n_nodes,), jnp.int32),
            pltpu.SemaphoreType.DMA,
            pltpu.SemaphoreType.DMA,
            pltpu.SemaphoreType.DMA,
            pltpu.SemaphoreType.DMA,
            pltpu.SemaphoreType.DMA,
        ],
    )
    def k(emb_h, idx_h, nidx_h, pidx_h, negidx_h, pos_out, neg_out,
          sn_v, sp_v, cn_v, cp_v, en0, en1, ep0, ep1, pb0, pb1, nb0, nb1,
          negi_v, idx_sp, sem_p, sem_a, sem_b, sem_o0, sem_o1):
        wid = lax.axis_index("s") * nc + lax.axis_index("c")
        base0 = wid * per_w
        row0 = wid * n_chunks

        ens = (en0, en1)
        eps = (ep0, ep1)
        pbs = (pb0, pb1)
        nbs = (nb0, nb1)
        sem_g = (sem_a, sem_b)
        sem_o = (sem_o0, sem_o1)

        @pl.when(lax.axis_index("s") == 0)
        def _():
            pltpu.sync_copy(idx_h, idx_sp)

        plsc.subcore_barrier()

        pltpu.sync_copy(negidx_h, negi_v)
        v = tuple(jnp.zeros((_LANES,), jnp.float32) for _ in range(k8))
        for h in range(neg_idx2d.shape[0]):
            pltpu.async_copy(idx_sp.at[negi_v.at[h]], cn_v.at[0], sem_p).wait()
            pltpu.async_copy(emb_h.at[cn_v.at[0]], en0, sem_p).wait()

            def vacc(j, vs):
                return tuple(
                    vs[kk] + en0[j, pl.ds(kk * _LANES, _LANES)]
                    for kk in range(k8)
                )

            v = lax.fori_loop(0, _C, vacc, v)

        pltpu.sync_copy(nidx_h.at[pl.ds(row0, n_chunks)], sn_v)
        pltpu.sync_copy(pidx_h.at[pl.ds(row0, n_chunks)], sp_v)

        def compose(j, carry):
            pltpu.make_async_copy(idx_sp.at[sn_v.at[j]], cn_v.at[j], sem_p).start()
            pltpu.make_async_copy(idx_sp.at[sp_v.at[j]], cp_v.at[j], sem_p).start()

            @pl.when(j >= lag)
            def _():
                pltpu.make_async_copy(
                    idx_sp.at[sn_v.at[0]], cn_v.at[0], sem_p).wait()
                pltpu.make_async_copy(
                    idx_sp.at[sp_v.at[0]], cp_v.at[0], sem_p).wait()

            return carry

        lax.fori_loop(0, n_chunks, compose, 0)
        for _ in range(lag):
            pltpu.make_async_copy(idx_sp.at[sn_v.at[0]], cn_v.at[0], sem_p).wait()
            pltpu.make_async_copy(idx_sp.at[sp_v.at[0]], cp_v.at[0], sem_p).wait()

        def fire_rows(g, b):
            pltpu.make_async_copy(emb_h.at[cn_v.at[g]], ens[b], sem_g[b]).start()
            pltpu.make_async_copy(emb_h.at[cp_v.at[g]], eps[b], sem_g[b]).start()

        def wait_rows(b):
            pltpu.make_async_copy(emb_h.at[cn_v.at[0]], ens[b], sem_g[b]).wait()
            pltpu.make_async_copy(emb_h.at[cp_v.at[0]], eps[b], sem_g[b]).wait()

        def drain_stores(b):
            pltpu.make_async_copy(
                pbs[b], pos_out.at[pl.ds(0, nrow)], sem_o[b]).wait()
            pltpu.make_async_copy(
                nbs[b], neg_out.at[pl.ds(0, nrow)], sem_o[b]).wait()

        def compute(g, b):
            en_v, ep_v = ens[b], eps[b]
            pb_v, nb_v = pbs[b], nbs[b]

            @plsc.parallel_loop(0, _C, 1, unroll=16)
            def row(j):
                e0 = en_v[j, pl.ds(0, _LANES)]
                p0 = ep_v[j, pl.ds(0, _LANES)]
                pos = e0 * p0
                neg = e0 * v[0]
                for kk in range(1, k8):
                    ek = en_v[j, pl.ds(kk * _LANES, _LANES)]
                    pk = ep_v[j, pl.ds(kk * _LANES, _LANES)]
                    pos = pos + ek * pk
                    neg = neg + ek * v[kk]
                jr = lax.shift_right_logical(j, 3)
                jc = lax.bitwise_and(j, 7) * _LANES
                pb_v[jr, pl.ds(jc, _LANES)] = pos
                nb_v[jr, pl.ds(jc, _LANES)] = neg
            base = pl.multiple_of((base0 + g * _C) * _LANES // 128, nrow)
            pltpu.make_async_copy(
                pb_v, pos_out.at[pl.ds(base, nrow)], sem_o[b]).start()
            pltpu.make_async_copy(
                nb_v, neg_out.at[pl.ds(base, nrow)], sem_o[b]).start()

        fire_rows(0, 0)

        def body2(t, carry):
            g0 = 2 * t
            fire_rows(g0 + 1, 1)
            wait_rows(0)

            @pl.when(t > 0)
            def _():
                drain_stores(0)

            compute(g0, 0)

            @pl.when(g0 + 2 < n_chunks)
            def _():
                fire_rows(g0 + 2, 0)

            wait_rows(1)

            @pl.when(t > 0)
            def _():
                drain_stores(1)

            compute(g0 + 1, 1)
            return carry

        lax.fori_loop(0, n_chunks // 2, body2, 0)
        drain_stores(0)
        drain_stores(1)

    return k(node_embedding, indices, nidx2d, pidx2d, neg_idx2d)


def _tc_loss(pos_partial, neg_partial, s):
    rows = pos_partial.shape[0]
    blk = 5120
    grid = rows // blk
    groups = 128 // _LANES

    def body(pref, nref, oref):
        i = pl.program_id(0)

        @pl.when(i == 0)
        def _():
            oref[...] = jnp.zeros((1, 1), jnp.float32)

        lane = lax.broadcasted_iota(jnp.int32, (128, groups), 0)
        grp = lax.broadcasted_iota(jnp.int32, (128, groups), 1)
        gmat = jnp.where(lane // _LANES == grp, 1.0, 0.0).astype(jnp.float32)

        ps = jax.lax.dot_general(
            pref[...], gmat, (((1,), (0,)), ((), ())),
            preferred_element_type=jnp.float32)
        ns = jax.lax.dot_general(
            nref[...], gmat, (((1,), (0,)), ((), ())),
            preferred_element_type=jnp.float32)
        acc = (3.0 * jnp.sum(jax.nn.softplus(-ps))
               + jnp.sum(jax.nn.softplus(ns)))
        oref[...] += jnp.reshape(acc * (1.0 / s), (1, 1))

    return pl.pallas_call(
        body,
        grid=(grid,),
        in_specs=[
            pl.BlockSpec((blk, 128), lambda i: (i, 0)),
            pl.BlockSpec((blk, 128), lambda i: (i, 0)),
        ],
        out_specs=pl.BlockSpec((1, 1), lambda i: (0, 0)),
        out_shape=jax.ShapeDtypeStruct((1, 1), jnp.float32),
    )(pos_partial, neg_partial)


@jax.jit
def kernel(node_embedding, indices, node_indices, pos_indices, neg_indices):
    nidx2d = node_indices.reshape(-1, _C)
    pidx2d = pos_indices.reshape(-1, _C)
    neg_idx2d = neg_indices.reshape(-1, _C)
    pos_p, neg_p = _sc_partials(
        node_embedding, indices, nidx2d, pidx2d, neg_idx2d)
    s = node_indices.shape[0]
    return _tc_loss(pos_p, neg_p, s).reshape(1)
